# Initial kernel scaffold; baseline (speedup 1.0000x reference)
import jax
import jax.numpy as jnp
from jax.experimental import pallas as pl


def kernel(lags, weather, lags_W1, lags_b1, bn_g, bn_b, lags_W2, lags_b2, wea_W1, wea_b1, wea_W2, wea_b2, mgn0_W, mgn0_b, mgn1_W, mgn1_b, reg_W, reg_b, src, dst):
    raise NotImplementedError("write your pallas kernel here")


# speedup vs baseline: 13.1212x; 13.1212x over previous
import jax
import jax.numpy as jnp
from jax.experimental import pallas as pl
from jax.experimental.pallas import tpu as pltpu

H = 512
BS = 32
NUM_LAGS = 9
N = BS * NUM_LAGS
E = 1024


def _fused_body(l_ref, w_ref, w1l_ref, b1l_ref, g_ref, bb_ref, w2l_ref,
                b2l_ref, w1w_ref, b1w_ref, w2w_ref, b2w_ref, m0w_ref,
                m0b_ref, m1w_ref, m1b_ref, regr_ref, regb_ref, src_ref,
                dst_ref, out_ref):
    f32 = jnp.float32
    dot = lambda a, b: jax.lax.dot_general(
        a, b, (((1,), (0,)), ((), ())), preferred_element_type=f32,
        precision=jax.lax.Precision.HIGHEST)

    ids = jax.lax.broadcasted_iota(jnp.int32, (E, N), 1)
    es = (src_ref[...] == ids).astype(f32)
    ed = (dst_ref[...] == ids).astype(f32)
    a = jax.lax.dot_general(ed, es, (((0,), (0,)), ((), ())),
                            preferred_element_type=f32,
                            precision=jax.lax.Precision.HIGHEST)
    deg = jnp.maximum(jnp.sum(a, axis=1, keepdims=True), 1.0)
    an = a / deg

    inv = 1.0 / jnp.sqrt(1.0 + 1e-5)
    l = jnp.tanh(l_ref[...] * w1l_ref[...] + b1l_ref[...])
    l = l * (g_ref[...] * inv) + bb_ref[...]
    l = jnp.tanh(dot(l, w2l_ref[...]) + b2l_ref[...])

    w = jnp.tanh(dot(w_ref[...], w1w_ref[...]) + b1w_ref[...])
    w = jnp.tanh(dot(w, w2w_ref[...]) + b2w_ref[...])

    ml = dot(an, l)
    mw = dot(an, w)
    l = dot(ml, m0w_ref[0]) + dot(mw, m0w_ref[1]) + m0b_ref[...]
    w = mw

    ml = dot(an, l)
    mw = dot(an, w)
    l = dot(ml, m1w_ref[0]) + dot(mw, m1w_ref[1]) + m1b_ref[...]

    s = jnp.sum(l * regr_ref[...], axis=1, keepdims=True)
    rows = jax.lax.broadcasted_iota(jnp.int32, (BS, N), 0)
    cols = jax.lax.broadcasted_iota(jnp.int32, (BS, N), 1)
    grp = (rows == cols // NUM_LAGS).astype(f32)
    out_ref[...] = dot(grp, s) + regb_ref[...]


def kernel(lags, weather, lags_W1, lags_b1, bn_g, bn_b, lags_W2, lags_b2,
           wea_W1, wea_b1, wea_W2, wea_b2, mgn0_W, mgn0_b, mgn1_W, mgn1_b,
           reg_W, reg_b, src, dst):
    f32 = jnp.float32
    l0 = lags.reshape(N, 1)
    w0 = weather.reshape(N, 8)
    reg_tiled = jnp.tile(reg_W.reshape(NUM_LAGS, H), (BS, 1))
    m0 = mgn0_W.reshape(2, H, H)
    m1 = mgn1_W.reshape(2, H, H)
    args = (
        l0, w0, lags_W1, lags_b1.reshape(1, H), bn_g.reshape(1, H),
        bn_b.reshape(1, H), lags_W2, lags_b2.reshape(1, H), wea_W1,
        wea_b1.reshape(1, H), wea_W2, wea_b2.reshape(1, H), m0,
        mgn0_b.reshape(1, H), m1, mgn1_b.reshape(1, H), reg_tiled,
        reg_b.reshape(1, 1), src.reshape(E, 1), dst.reshape(E, 1),
    )
    return pl.pallas_call(
        _fused_body,
        out_shape=jax.ShapeDtypeStruct((BS, 1), f32),
    )(*args)
